# baseline (speedup 1.0000x reference)
import functools

import jax
import jax.numpy as jnp
from jax import lax
from jax.experimental import pallas as pl
from jax.experimental.pallas import tpu as pltpu
from jax.experimental.pallas import tpu_sc as plsc

_NW = 32
_B = 16384 * 32
_D = 128
_V = 32
_G = 128
_L = 16
_PER_W = _B // _NW
_NGRP = _PER_W // _G

_mesh = plsc.VectorSubcoreMesh(core_axis_name="c", subcore_axis_name="s")


@functools.partial(
    pl.kernel,
    mesh=_mesh,
    out_type=jax.ShapeDtypeStruct((_B * _D,), jnp.float32),
    compiler_params=pltpu.CompilerParams(needs_layout_passes=False),
    scratch_types=[
        pltpu.VMEM((_PER_W,), jnp.int32),
        pltpu.VMEM((_V * _D,), jnp.float32),
        pltpu.VMEM((_G * _D,), jnp.float32),
        pltpu.VMEM((_G * _D,), jnp.float32),
        pltpu.SemaphoreType.DMA,
        pltpu.SemaphoreType.DMA,
    ],
)
def _gather_all(idx_hbm, table_hbm, out_hbm, idx_v, tab_v, b0, b1, w0, w1):
    wid = lax.axis_index("s") * 2 + lax.axis_index("c")
    base = wid * _PER_W
    pltpu.sync_copy(idx_hbm.at[wid], idx_v)
    pltpu.sync_copy(table_hbm, tab_v)

    bufs = (b0, b1)
    wsems = (w0, w1)
    lanes = lax.iota(jnp.int32, _L)

    def w_start(p, g):
        pltpu.async_copy(bufs[p], out_hbm.at[pl.ds((base + g * _G) * _D,
                                                   _G * _D)], wsems[p])

    def w_wait(p):
        pltpu.make_async_copy(bufs[p], out_hbm.at[pl.ds(base * _D, _G * _D)],
                              wsems[p]).wait()

    def build(g, buf):
        @plsc.parallel_loop(0, _G // _L, unroll=1)
        def sgbody(r16):
            src16 = idx_v[pl.ds(g * _G + r16 * _L, _L)] * _D
            dst16 = r16 * _L * _D
            for l in range(_L):
                addr = jnp.take_along_axis(
                    src16, jnp.full((_L,), l, jnp.int32), axis=0) + lanes
                dst = dst16 + l * _D
                for c0 in range(_D // _L):
                    buf[pl.ds(dst + c0 * _L, _L)] = (
                        plsc.load_gather(tab_v, [addr + c0 * _L]))

    build(0, bufs[0])
    w_start(0, 0)
    build(1, bufs[1])
    w_start(1, 1)

    def body(t, carry):
        g = 2 * t
        w_wait(0)
        build(g, bufs[0])
        w_start(0, g)
        w_wait(1)
        build(g + 1, bufs[1])
        w_start(1, g + 1)
        return carry

    lax.fori_loop(1, _NGRP // 2, body, 0)
    w_wait(0)
    w_wait(1)


def kernel(x, weight):
    xr = x.reshape(_NW, _PER_W)
    out = _gather_all(xr, weight.reshape(_V * _D))
    return out.reshape(16384, 32, _D)


# speedup vs baseline: 2.1802x; 2.1802x over previous
import functools

import jax
import jax.numpy as jnp
from jax import lax
from jax.experimental import pallas as pl
from jax.experimental.pallas import tpu as pltpu
from jax.experimental.pallas import tpu_sc as plsc

_NW = 32
_B = 16384 * 32
_D = 128
_V = 32
_G = 128
_PER_W = _B // _NW
_NGRP = _PER_W // _G
_NT = _NGRP // 4

_mesh = plsc.VectorSubcoreMesh(core_axis_name="c", subcore_axis_name="s")


@functools.partial(
    pl.kernel,
    mesh=_mesh,
    out_type=jax.ShapeDtypeStruct((_B, _D), jnp.float32),
    compiler_params=pltpu.CompilerParams(needs_layout_passes=False),
    scratch_types=[
        pltpu.VMEM((_NGRP, _G), jnp.int32),
        pltpu.VMEM((_G, _D), jnp.float32),
        pltpu.VMEM((_G, _D), jnp.float32),
        pltpu.VMEM((_G, _D), jnp.float32),
        pltpu.VMEM((_G, _D), jnp.float32),
        pltpu.VMEM_SHARED((_V, _D), jnp.float32),
        pltpu.SemaphoreType.DMA,
        pltpu.SemaphoreType.DMA,
        pltpu.SemaphoreType.DMA,
        pltpu.SemaphoreType.DMA,
        pltpu.SemaphoreType.DMA,
        pltpu.SemaphoreType.DMA,
        pltpu.SemaphoreType.DMA,
        pltpu.SemaphoreType.DMA,
    ],
)
def _gather_all(idx_hbm, table_hbm, out_hbm, idx_v, b0, b1, b2, b3, tab_sh,
                g0, g1, g2, g3, w0, w1, w2, w3):
    sid = lax.axis_index("s")
    wid = sid * 2 + lax.axis_index("c")
    base = wid * _PER_W

    @pl.when(sid == 0)
    def _():
        pltpu.sync_copy(table_hbm, tab_sh)

    pltpu.sync_copy(idx_hbm.at[wid], idx_v)
    plsc.subcore_barrier()

    bufs = (b0, b1, b2, b3)
    gsems = (g0, g1, g2, g3)
    wsems = (w0, w1, w2, w3)

    def g_start(b, g):
        pltpu.async_copy(tab_sh.at[idx_v.at[g]], bufs[b], gsems[b])

    def g_wait(b):
        pltpu.make_async_copy(tab_sh.at[idx_v.at[0]], bufs[b],
                              gsems[b]).wait()

    def w_start(b, g):
        pltpu.async_copy(bufs[b], out_hbm.at[pl.ds(base + g * _G, _G)],
                         wsems[b])

    def w_wait(b):
        pltpu.make_async_copy(bufs[b], out_hbm.at[pl.ds(base, _G)],
                              wsems[b]).wait()

    g_start(0, 0)
    g_start(1, 1)
    for i, b in enumerate((0, 1)):
        g_wait(b)
        w_start(b, i)
    g_start(2, 2)
    g_start(3, 3)
    for i, b in enumerate((2, 3)):
        g_wait(b)
        w_start(b, 2 + i)
    for i, b in enumerate((0, 1)):
        w_wait(b)
        g_start(b, 4 + i)

    def body(t, carry):
        ga = 4 * t
        for i, b in enumerate((0, 1)):
            g_wait(b)
            w_start(b, ga + i)
        for i, b in enumerate((2, 3)):
            w_wait(b)
            g_start(b, ga + 2 + i)
        for i, b in enumerate((2, 3)):
            g_wait(b)
            w_start(b, ga + 2 + i)
        for i, b in enumerate((0, 1)):
            w_wait(b)
            g_start(b, ga + 4 + i)
        return carry

    lax.fori_loop(1, _NT - 1, body, 0)

    ga = 4 * (_NT - 1)
    for i, b in enumerate((0, 1)):
        g_wait(b)
        w_start(b, ga + i)
    for i, b in enumerate((2, 3)):
        w_wait(b)
        g_start(b, ga + 2 + i)
    for i, b in enumerate((2, 3)):
        g_wait(b)
        w_start(b, ga + 2 + i)
    for b in (0, 1, 2, 3):
        w_wait(b)


def kernel(x, weight):
    xr = x.reshape(_NW, _NGRP, _G)
    out = _gather_all(xr, weight)
    return out.reshape(16384, 32, _D)
